# baseline (speedup 1.0000x reference)
import jax
import jax.numpy as jnp
from jax.experimental import pallas as pl
from jax.experimental.pallas import tpu as pltpu
from jax.experimental.pallas import tpu_sc as plsc

B = 4
SEQ = 2048
HIDDEN = 128
N_CHUNKS = 2
B_CHUNK = B // N_CHUNKS
ROWS_CHUNK = B_CHUNK * SEQ

_GATHER_WINDOW = 128


def _sc_gather(tok_table, flat_ids):
    mesh = plsc.VectorSubcoreMesh(core_axis_name="core",
                                  subcore_axis_name="subcore")
    n = flat_ids.shape[1]

    @pl.kernel(out_type=jax.ShapeDtypeStruct((n, HIDDEN), jnp.float32),
               mesh=mesh)
    def gather_kernel(tab_hbm, ids_hbm, out_hbm):
        def body(ids_vmem, out_vmem):
            pltpu.sync_copy(tab_hbm.at[ids_vmem.at[0]], out_vmem)

        pltpu.emit_pipeline(
            body,
            grid=(n // _GATHER_WINDOW,),
            in_specs=[pl.BlockSpec((1, _GATHER_WINDOW),
                                   index_map=lambda i: (0, i))],
            out_specs=[pl.BlockSpec((_GATHER_WINDOW, HIDDEN),
                                    index_map=lambda i: (i, 0))],
            core_axis_name=("core", "subcore"),
            dimension_semantics=(pltpu.PARALLEL,),
        )(ids_hbm, out_hbm)

    return gather_kernel(tok_table, flat_ids)


def _tc_dense(tok_emb, seg_ids3, pos_table, seg_table, gamma2, beta2):
    b = tok_emb.shape[0]

    def body(x_ref, sid_ref, pos_ref, segtab_ref, gamma_ref, beta_ref, o_ref):
        x = x_ref[0] + pos_ref[...]
        sid = sid_ref[0, 0]
        seg = jnp.where((sid[:, None]) == 0,
                        segtab_ref[0:1, :], segtab_ref[1:2, :])
        x = x + seg
        mu = jnp.mean(x, axis=-1, keepdims=True)
        var = jnp.mean((x - mu) ** 2, axis=-1, keepdims=True)
        xhat = (x - mu) * jax.lax.rsqrt(var + 1e-5)
        o_ref[0] = xhat * gamma_ref[...] + beta_ref[...]

    return pl.pallas_call(
        body,
        grid=(b,),
        in_specs=[
            pl.BlockSpec((1, SEQ, HIDDEN), lambda i: (i, 0, 0)),
            pl.BlockSpec((1, 1, SEQ), lambda i: (i, 0, 0)),
            pl.BlockSpec((SEQ, HIDDEN), lambda i: (0, 0)),
            pl.BlockSpec((2, HIDDEN), lambda i: (0, 0)),
            pl.BlockSpec((1, HIDDEN), lambda i: (0, 0)),
            pl.BlockSpec((1, HIDDEN), lambda i: (0, 0)),
        ],
        out_specs=pl.BlockSpec((1, SEQ, HIDDEN), lambda i: (i, 0, 0)),
        out_shape=jax.ShapeDtypeStruct((b, SEQ, HIDDEN), jnp.float32),
    )(tok_emb, seg_ids3, pos_table, seg_table, gamma2, beta2)


def kernel(token_ids, seg_ids, tok_table, pos_table, seg_table, gamma, beta):
    ids = token_ids.astype(jnp.int32)
    sids = seg_ids.astype(jnp.int32)
    gamma2 = gamma.reshape(1, HIDDEN)
    beta2 = beta.reshape(1, HIDDEN)
    outs = []
    for c in range(N_CHUNKS):
        lo = c * B_CHUNK
        flat_ids = ids[lo:lo + B_CHUNK].reshape(1, ROWS_CHUNK)
        tok_emb = _sc_gather(tok_table, flat_ids).reshape(B_CHUNK, SEQ, HIDDEN)
        sid3 = sids[lo:lo + B_CHUNK].reshape(B_CHUNK, 1, SEQ)
        outs.append(_tc_dense(tok_emb, sid3, pos_table, seg_table,
                              gamma2, beta2))
    return jnp.concatenate(outs, axis=0)


# speedup vs baseline: 1.2251x; 1.2251x over previous
import functools

import jax
import jax.numpy as jnp
from jax import lax
from jax.experimental import pallas as pl
from jax.experimental.pallas import tpu as pltpu
from jax.experimental.pallas import tpu_sc as plsc

B = 4
SEQ = 2048
HIDDEN = 128
N_ROWS = B * SEQ
N_WORKERS = 32
ROWS_PER_WORKER = N_ROWS // N_WORKERS


def _sc_gather(tok_table, flat_ids):
    mesh = plsc.VectorSubcoreMesh(core_axis_name="c", subcore_axis_name="s")

    @functools.partial(
        pl.kernel, mesh=mesh,
        out_type=jax.ShapeDtypeStruct((N_ROWS, HIDDEN), jnp.float32),
        scratch_types=[
            pltpu.VMEM((ROWS_PER_WORKER,), jnp.int32),
            pltpu.VMEM((ROWS_PER_WORKER, HIDDEN), jnp.float32),
            pltpu.SemaphoreType.DMA,
        ],
    )
    def gather_kernel(tab_hbm, idx_hbm, out_hbm, idx_v, rows_v, sem):
        wid = lax.axis_index("s") * 2 + lax.axis_index("c")
        base = wid * ROWS_PER_WORKER
        pltpu.sync_copy(idx_hbm.at[pl.ds(base, ROWS_PER_WORKER)], idx_v)
        pltpu.async_copy(tab_hbm.at[idx_v], rows_v, sem).wait()
        pltpu.sync_copy(rows_v, out_hbm.at[pl.ds(base, ROWS_PER_WORKER)])

    return gather_kernel(tok_table, flat_ids)


def _tc_dense(tok_emb, seg_ids3, pos_table, seg_table, gamma2, beta2):
    def body(x_ref, sid_ref, pos_ref, segtab_ref, gamma_ref, beta_ref, o_ref):
        x = x_ref[0] + pos_ref[...]
        sid = sid_ref[0, 0]
        seg = jnp.where((sid[:, None]) == 0,
                        segtab_ref[0:1, :], segtab_ref[1:2, :])
        x = x + seg
        mu = jnp.mean(x, axis=-1, keepdims=True)
        var = jnp.mean((x - mu) ** 2, axis=-1, keepdims=True)
        xhat = (x - mu) * jax.lax.rsqrt(var + 1e-5)
        o_ref[0] = xhat * gamma_ref[...] + beta_ref[...]

    return pl.pallas_call(
        body,
        grid=(B,),
        in_specs=[
            pl.BlockSpec((1, SEQ, HIDDEN), lambda b: (b, 0, 0)),
            pl.BlockSpec((1, 1, SEQ), lambda b: (b, 0, 0)),
            pl.BlockSpec((SEQ, HIDDEN), lambda b: (0, 0)),
            pl.BlockSpec((2, HIDDEN), lambda b: (0, 0)),
            pl.BlockSpec((1, HIDDEN), lambda b: (0, 0)),
            pl.BlockSpec((1, HIDDEN), lambda b: (0, 0)),
        ],
        out_specs=pl.BlockSpec((1, SEQ, HIDDEN), lambda b: (b, 0, 0)),
        out_shape=jax.ShapeDtypeStruct((B, SEQ, HIDDEN), jnp.float32),
    )(tok_emb, seg_ids3, pos_table, seg_table, gamma2, beta2)


def kernel(token_ids, seg_ids, tok_table, pos_table, seg_table, gamma, beta):
    flat_ids = token_ids.astype(jnp.int32).reshape(N_ROWS)
    tok_emb = _sc_gather(tok_table, flat_ids).reshape(B, SEQ, HIDDEN)
    seg_ids3 = seg_ids.astype(jnp.int32).reshape(B, 1, SEQ)
    gamma2 = gamma.reshape(1, HIDDEN)
    beta2 = beta.reshape(1, HIDDEN)
    return _tc_dense(tok_emb, seg_ids3, pos_table, seg_table, gamma2, beta2)
